# baseline (speedup 1.0000x reference)
import functools
import math

import jax
import jax.numpy as jnp
from jax import lax
from jax.experimental import pallas as pl
from jax.experimental.pallas import tpu as pltpu
from jax.experimental.pallas import tpu_sc as plsc

D_MODEL = 1024
SCALE = math.sqrt(D_MODEL)
NC, NS, LANES = 2, 16, 16
NW = NC * NS
CHUNK = 32
NBUF = 3


def _embed_kernel(n_per_w, seq_len, table_hbm, x_hbm, out_hbm, idx_v, rows_v,
                  gsem, wsem):
    wid = lax.axis_index("s") * NC + lax.axis_index("c")
    base = wid * n_per_w
    w_per_row = seq_len // n_per_w
    b = wid // w_per_row
    off = (wid % w_per_row) * n_per_w
    n_chunks = n_per_w // CHUNK

    def gather(c, nb):
        pltpu.async_copy(
            table_hbm.at[idx_v.at[pl.ds(c * CHUNK, CHUNK)]],
            rows_v.at[nb], gsem[nb])

    def wait_gather(nb):
        pltpu.make_async_copy(
            table_hbm.at[idx_v.at[pl.ds(0, CHUNK)]],
            rows_v.at[nb], gsem[nb]).wait()

    def writeback(c, nb):
        pltpu.async_copy(
            rows_v.at[nb], out_hbm.at[pl.ds(base + c * CHUNK, CHUNK)],
            wsem[nb])

    def wait_writeback(c, nb):
        pltpu.make_async_copy(
            rows_v.at[nb], out_hbm.at[pl.ds(base + c * CHUNK, CHUNK)],
            wsem[nb]).wait()

    hw = n_per_w // 2
    pltpu.sync_copy(x_hbm.at[b, pl.ds(off, hw)], idx_v.at[pl.ds(0, hw)])
    for c in range(NBUF - 1):
        gather(c, c)
    pltpu.sync_copy(x_hbm.at[b, pl.ds(off + hw, hw)],
                    idx_v.at[pl.ds(hw, hw)])
    for c in range(n_chunks):
        nb = c % NBUF
        if c + NBUF - 1 < n_chunks:
            if c >= 1:
                wait_writeback(c - 1, (c - 1) % NBUF)
            gather(c + NBUF - 1, (c + NBUF - 1) % NBUF)
        wait_gather(nb)

        @plsc.parallel_loop(0, CHUNK, unroll=2)
        def row_body(r):
            for v in range(D_MODEL // LANES):
                sl = pl.ds(v * LANES, LANES)
                rows_v[nb, r, sl] = rows_v[nb, r, sl] * SCALE

        writeback(c, nb)

    for c in range(max(0, n_chunks - NBUF), n_chunks):
        wait_writeback(c, c % NBUF)


@jax.jit
def kernel(x, table):
    B, L = x.shape
    n = B * L
    n_per_w = n // NW

    mesh = plsc.VectorSubcoreMesh(
        core_axis_name="c", subcore_axis_name="s", num_cores=NC, num_subcores=NS
    )
    out = pl.kernel(
        functools.partial(_embed_kernel, n_per_w, L),
        out_type=jax.ShapeDtypeStruct((n, D_MODEL), jnp.float32),
        mesh=mesh,
        scratch_types=[
            pltpu.VMEM((n_per_w,), jnp.int32),
            pltpu.VMEM((NBUF, CHUNK, D_MODEL), jnp.float32),
            [pltpu.SemaphoreType.DMA] * NBUF,
            [pltpu.SemaphoreType.DMA] * NBUF,
        ],
    )(table, x.astype(jnp.int32))
    return out.reshape(B, L, D_MODEL)


# speedup vs baseline: 1.1594x; 1.1594x over previous
import functools
import math

import jax
import jax.numpy as jnp
from jax import lax
from jax.experimental import pallas as pl
from jax.experimental.pallas import tpu as pltpu
from jax.experimental.pallas import tpu_sc as plsc

D_MODEL = 1024
SCALE = math.sqrt(D_MODEL)
NC, NS, LANES = 2, 16, 16
NW = NC * NS
CHUNK = 32
NBUF = 3


def _embed_kernel(n_per_w, seq_len, table_hbm, x_hbm, out_hbm, idx_v, rows_v,
                  gsem, wsem):
    wid = lax.axis_index("s") * NC + lax.axis_index("c")
    base = wid * n_per_w
    w_per_row = seq_len // n_per_w
    b = wid // w_per_row
    off = (wid % w_per_row) * n_per_w
    n_chunks = n_per_w // CHUNK

    def gather(c, nb):
        pltpu.async_copy(
            table_hbm.at[idx_v.at[pl.ds(c * CHUNK, CHUNK)]],
            rows_v.at[nb], gsem[nb])

    def wait_gather(nb):
        pltpu.make_async_copy(
            table_hbm.at[idx_v.at[pl.ds(0, CHUNK)]],
            rows_v.at[nb], gsem[nb]).wait()

    def writeback(c, nb):
        pltpu.async_copy(
            rows_v.at[nb], out_hbm.at[pl.ds(base + c * CHUNK, CHUNK)],
            wsem[nb])

    def wait_writeback(c, nb):
        pltpu.make_async_copy(
            rows_v.at[nb], out_hbm.at[pl.ds(base + c * CHUNK, CHUNK)],
            wsem[nb]).wait()

    hw = n_per_w // 2
    pltpu.sync_copy(x_hbm.at[b, pl.ds(off, hw)], idx_v.at[pl.ds(0, hw)])
    for c in range(NBUF - 1):
        gather(c, c)
    pltpu.sync_copy(x_hbm.at[b, pl.ds(off + hw, hw)],
                    idx_v.at[pl.ds(hw, hw)])
    for c in range(n_chunks):
        nb = c % NBUF
        if c + NBUF - 1 < n_chunks:
            if c >= 1:
                wait_writeback(c - 1, (c - 1) % NBUF)
            gather(c + NBUF - 1, (c + NBUF - 1) % NBUF)
        wait_gather(nb)

        @plsc.parallel_loop(0, CHUNK)
        def row_body(r):
            for v in range(D_MODEL // LANES):
                sl = pl.ds(v * LANES, LANES)
                rows_v[nb, r, sl] = rows_v[nb, r, sl] * SCALE

        writeback(c, nb)

    for c in range(max(0, n_chunks - NBUF), n_chunks):
        wait_writeback(c, c % NBUF)


@jax.jit
def kernel(x, table):
    B, L = x.shape
    n = B * L
    n_per_w = n // NW

    mesh = plsc.VectorSubcoreMesh(
        core_axis_name="c", subcore_axis_name="s", num_cores=NC, num_subcores=NS
    )
    out = pl.kernel(
        functools.partial(_embed_kernel, n_per_w, L),
        out_type=jax.ShapeDtypeStruct((n, D_MODEL), jnp.float32),
        mesh=mesh,
        scratch_types=[
            pltpu.VMEM((n_per_w,), jnp.int32),
            pltpu.VMEM((NBUF, CHUNK, D_MODEL), jnp.float32),
            [pltpu.SemaphoreType.DMA] * NBUF,
            [pltpu.SemaphoreType.DMA] * NBUF,
        ],
    )(table, x.astype(jnp.int32))
    return out.reshape(B, L, D_MODEL)


# speedup vs baseline: 1.2203x; 1.0525x over previous
import functools
import math

import jax
import jax.numpy as jnp
from jax import lax
from jax.experimental import pallas as pl
from jax.experimental.pallas import tpu as pltpu
from jax.experimental.pallas import tpu_sc as plsc

D_MODEL = 1024
SCALE = math.sqrt(D_MODEL)
NC, NS, LANES = 2, 16, 16
NW = NC * NS
CHUNK = 32
NBUF = 3


def _embed_kernel(n_per_w, seq_len, table_hbm, x_hbm, out_hbm, idx_v, rows_v,
                  gsem, wsem):
    wid = lax.axis_index("s") * NC + lax.axis_index("c")
    base = wid * n_per_w
    w_per_row = seq_len // n_per_w
    b = wid // w_per_row
    off = (wid % w_per_row) * n_per_w

    n_chunks = n_per_w // CHUNK

    def gather(c, nb):
        start = pl.multiple_of(c * CHUNK, CHUNK)
        pltpu.async_copy(
            table_hbm.at[idx_v.at[pl.ds(start, CHUNK)]],
            rows_v.at[nb], gsem.at[nb])

    def wait_gather(nb):
        pltpu.make_async_copy(
            table_hbm.at[idx_v.at[pl.ds(0, CHUNK)]],
            rows_v.at[nb], gsem.at[nb]).wait()

    def writeback(c, nb):
        start = pl.multiple_of(base + c * CHUNK, CHUNK)
        pltpu.async_copy(
            rows_v.at[nb], out_hbm.at[pl.ds(start, CHUNK)],
            wsem.at[nb])

    def wait_writeback(c, nb):
        start = pl.multiple_of(base + c * CHUNK, CHUNK)
        pltpu.make_async_copy(
            rows_v.at[nb], out_hbm.at[pl.ds(start, CHUNK)],
            wsem.at[nb]).wait()

    def scale(nb):
        @plsc.parallel_loop(0, CHUNK)
        def row_body(r):
            for v in range(D_MODEL // LANES):
                sl = pl.ds(v * LANES, LANES)
                rows_v[nb, r, sl] = rows_v[nb, r, sl] * SCALE

    hw = n_per_w // 2
    pltpu.sync_copy(x_hbm.at[b, pl.ds(off, hw)], idx_v.at[pl.ds(0, hw)])
    gather(0, 0)
    gather(1, 1)
    pltpu.sync_copy(x_hbm.at[b, pl.ds(off + hw, hw)],
                    idx_v.at[pl.ds(hw, hw)])

    def chunk_body(c, _):
        nb = lax.rem(c, NBUF)
        pb = lax.rem(c + NBUF - 1, NBUF)

        @pl.when(c >= 1)
        def _():
            wait_writeback(c - 1, pb)

        gather(c + NBUF - 1, pb)
        wait_gather(nb)
        scale(nb)
        writeback(c, nb)
        return 0

    lax.fori_loop(0, n_chunks - NBUF + 1, chunk_body, 0)

    for c in range(n_chunks - NBUF + 1, n_chunks):
        nb = c % NBUF
        wait_gather(nb)
        scale(nb)
        writeback(c, nb)

    for c in range(n_chunks - NBUF, n_chunks):
        wait_writeback(c, c % NBUF)


@jax.jit
def kernel(x, table):
    B, L = x.shape
    n = B * L
    n_per_w = n // NW

    mesh = plsc.VectorSubcoreMesh(
        core_axis_name="c", subcore_axis_name="s", num_cores=NC, num_subcores=NS
    )
    out = pl.kernel(
        functools.partial(_embed_kernel, n_per_w, L),
        out_type=jax.ShapeDtypeStruct((n, D_MODEL), jnp.float32),
        mesh=mesh,
        scratch_types=[
            pltpu.VMEM((n_per_w,), jnp.int32),
            pltpu.VMEM((NBUF, CHUNK, D_MODEL), jnp.float32),
            pltpu.SemaphoreType.DMA((NBUF,)),
            pltpu.SemaphoreType.DMA((NBUF,)),
        ],
    )(table, x.astype(jnp.int32))
    return out.reshape(B, L, D_MODEL)


# speedup vs baseline: 1.2452x; 1.0204x over previous
import functools
import math

import jax
import jax.numpy as jnp
from jax import lax
from jax.experimental import pallas as pl
from jax.experimental.pallas import tpu as pltpu
from jax.experimental.pallas import tpu_sc as plsc

D_MODEL = 1024
SCALE = math.sqrt(D_MODEL)
NC, NS, LANES = 2, 16, 16
NW = NC * NS
CHUNK = 32
NBUF = 3


def _embed_kernel(n_per_w, seq_len, table_hbm, x_hbm, out_hbm, idx_v, rows_v,
                  gsem, wsem):
    wid = lax.axis_index("s") * NC + lax.axis_index("c")
    base = wid * n_per_w
    w_per_row = seq_len // n_per_w
    b = wid // w_per_row
    off = (wid % w_per_row) * n_per_w

    n_chunks = n_per_w // CHUNK

    def gather(c, nb):
        start = pl.multiple_of(c * CHUNK, CHUNK)
        pltpu.async_copy(
            table_hbm.at[idx_v.at[pl.ds(start, CHUNK)]],
            rows_v.at[nb], gsem.at[nb])

    def wait_gather(nb):
        pltpu.make_async_copy(
            table_hbm.at[idx_v.at[pl.ds(0, CHUNK)]],
            rows_v.at[nb], gsem.at[nb]).wait()

    def writeback(c, nb):
        start = pl.multiple_of(base + c * CHUNK, CHUNK)
        pltpu.async_copy(
            rows_v.at[nb], out_hbm.at[pl.ds(start, CHUNK)],
            wsem.at[nb])

    def wait_writeback(c, nb):
        start = pl.multiple_of(base + c * CHUNK, CHUNK)
        pltpu.make_async_copy(
            rows_v.at[nb], out_hbm.at[pl.ds(start, CHUNK)],
            wsem.at[nb]).wait()

    def scale(nb):
        @plsc.parallel_loop(0, CHUNK)
        def row_body(r):
            for v in range(D_MODEL // LANES):
                sl = pl.ds(v * LANES, LANES)
                rows_v[nb, r, sl] = rows_v[nb, r, sl] * SCALE

    hw = n_per_w // 2
    pltpu.sync_copy(x_hbm.at[b, pl.ds(off, hw)], idx_v.at[pl.ds(0, hw)])
    gather(0, 0)
    gather(1, 1)
    pltpu.sync_copy(x_hbm.at[b, pl.ds(off + hw, hw)],
                    idx_v.at[pl.ds(hw, hw)])

    def chunk_body(c, _):
        nb = lax.rem(c, NBUF)
        pb = lax.rem(c + NBUF - 1, NBUF)
        prefetch = c + NBUF - 1 < n_chunks

        @pl.when(jnp.logical_and(c >= 1, prefetch))
        def _():
            wait_writeback(c - 1, pb)

        @pl.when(prefetch)
        def _():
            gather(c + NBUF - 1, pb)

        wait_gather(nb)
        scale(nb)
        writeback(c, nb)
        return 0

    lax.fori_loop(0, n_chunks, chunk_body, 0)

    for c in range(n_chunks - NBUF, n_chunks):
        wait_writeback(c, c % NBUF)


@jax.jit
def kernel(x, table):
    B, L = x.shape
    n = B * L
    n_per_w = n // NW

    mesh = plsc.VectorSubcoreMesh(
        core_axis_name="c", subcore_axis_name="s", num_cores=NC, num_subcores=NS
    )
    out = pl.kernel(
        functools.partial(_embed_kernel, n_per_w, L),
        out_type=jax.ShapeDtypeStruct((n, D_MODEL), jnp.float32),
        mesh=mesh,
        scratch_types=[
            pltpu.VMEM((n_per_w,), jnp.int32),
            pltpu.VMEM((NBUF, CHUNK, D_MODEL), jnp.float32),
            pltpu.SemaphoreType.DMA((NBUF,)),
            pltpu.SemaphoreType.DMA((NBUF,)),
        ],
    )(table, x.astype(jnp.int32))
    return out.reshape(B, L, D_MODEL)


# speedup vs baseline: 1.2545x; 1.0075x over previous
import functools
import math

import jax
import jax.numpy as jnp
from jax import lax
from jax.experimental import pallas as pl
from jax.experimental.pallas import tpu as pltpu
from jax.experimental.pallas import tpu_sc as plsc

D_MODEL = 1024
SCALE = math.sqrt(D_MODEL)
NC, NS, LANES = 2, 16, 16
NW = NC * NS
CHUNK = 32
NBUF = 3


def _embed_kernel(n_per_w, seq_len, table_hbm, x_hbm, out_hbm, idx_v, rows_v,
                  gsem, wsem):
    wid = lax.axis_index("s") * NC + lax.axis_index("c")
    base = wid * n_per_w
    w_per_row = seq_len // n_per_w
    b = wid // w_per_row
    off = (wid % w_per_row) * n_per_w

    n_chunks = n_per_w // CHUNK

    def gather(c, nb):
        start = pl.multiple_of(c * CHUNK, CHUNK)
        pltpu.async_copy(
            table_hbm.at[idx_v.at[pl.ds(start, CHUNK)]],
            rows_v.at[nb], gsem.at[nb])

    def wait_gather(nb):
        pltpu.make_async_copy(
            table_hbm.at[idx_v.at[pl.ds(0, CHUNK)]],
            rows_v.at[nb], gsem.at[nb]).wait()

    def writeback(c, nb):
        start = pl.multiple_of(base + c * CHUNK, CHUNK)
        pltpu.async_copy(
            rows_v.at[nb], out_hbm.at[pl.ds(start, CHUNK)],
            wsem.at[nb])

    def wait_writeback(c, nb):
        start = pl.multiple_of(base + c * CHUNK, CHUNK)
        pltpu.make_async_copy(
            rows_v.at[nb], out_hbm.at[pl.ds(start, CHUNK)],
            wsem.at[nb]).wait()

    def scale(nb):
        @plsc.parallel_loop(0, CHUNK)
        def row_body(r):
            @plsc.parallel_loop(0, D_MODEL, LANES, unroll=8)
            def col_body(v):
                sl = pl.ds(v, LANES)
                rows_v[nb, r, sl] = rows_v[nb, r, sl] * SCALE

    hw = n_per_w // 2
    pltpu.sync_copy(x_hbm.at[b, pl.ds(off, hw)], idx_v.at[pl.ds(0, hw)])
    gather(0, 0)
    gather(1, 1)
    pltpu.sync_copy(x_hbm.at[b, pl.ds(off + hw, hw)],
                    idx_v.at[pl.ds(hw, hw)])

    def chunk_body(c, _):
        nb = lax.rem(c, NBUF)
        pb = lax.rem(c + NBUF - 1, NBUF)
        prefetch = c + NBUF - 1 < n_chunks

        @pl.when(jnp.logical_and(c >= 1, prefetch))
        def _():
            wait_writeback(c - 1, pb)

        @pl.when(prefetch)
        def _():
            gather(c + NBUF - 1, pb)

        wait_gather(nb)
        scale(nb)
        writeback(c, nb)
        return 0

    lax.fori_loop(0, n_chunks, chunk_body, 0)

    for c in range(n_chunks - NBUF, n_chunks):
        wait_writeback(c, c % NBUF)


@jax.jit
def kernel(x, table):
    B, L = x.shape
    n = B * L
    n_per_w = n // NW

    mesh = plsc.VectorSubcoreMesh(
        core_axis_name="c", subcore_axis_name="s", num_cores=NC, num_subcores=NS
    )
    out = pl.kernel(
        functools.partial(_embed_kernel, n_per_w, L),
        out_type=jax.ShapeDtypeStruct((n, D_MODEL), jnp.float32),
        mesh=mesh,
        scratch_types=[
            pltpu.VMEM((n_per_w,), jnp.int32),
            pltpu.VMEM((NBUF, CHUNK, D_MODEL), jnp.float32),
            pltpu.SemaphoreType.DMA((NBUF,)),
            pltpu.SemaphoreType.DMA((NBUF,)),
        ],
    )(table, x.astype(jnp.int32))
    return out.reshape(B, L, D_MODEL)


# speedup vs baseline: 1.2680x; 1.0108x over previous
import functools
import math

import jax
import jax.numpy as jnp
from jax import lax
from jax.experimental import pallas as pl
from jax.experimental.pallas import tpu as pltpu
from jax.experimental.pallas import tpu_sc as plsc

D_MODEL = 1024
SCALE = math.sqrt(D_MODEL)
NC, NS, LANES = 2, 16, 16
NW = NC * NS
FIRST = 16
CHUNK = 24
NBUF = 3


def _scale_rows(ref, nb, n_rows):
    @plsc.parallel_loop(0, n_rows)
    def row_body(r):
        @plsc.parallel_loop(0, D_MODEL, LANES, unroll=8)
        def col_body(v):
            sl = pl.ds(v, LANES)
            if nb is None:
                ref[r, sl] = ref[r, sl] * SCALE
            else:
                ref[nb, r, sl] = ref[nb, r, sl] * SCALE


def _embed_kernel(n_per_w, seq_len, table_hbm, x_hbm, out_hbm, idx_v,
                  first_v, rows_v, gsem0, wsem0, gsem, wsem):
    wid = lax.axis_index("s") * NC + lax.axis_index("c")
    base = wid * n_per_w
    w_per_row = seq_len // n_per_w
    b = wid // w_per_row
    off = (wid % w_per_row) * n_per_w

    n_ring = (n_per_w - FIRST) // CHUNK

    def gather(c, nb):
        start = pl.multiple_of(FIRST + c * CHUNK, 8)
        pltpu.async_copy(
            table_hbm.at[idx_v.at[pl.ds(start, CHUNK)]],
            rows_v.at[nb], gsem.at[nb])

    def wait_gather(nb):
        pltpu.make_async_copy(
            table_hbm.at[idx_v.at[pl.ds(0, CHUNK)]],
            rows_v.at[nb], gsem.at[nb]).wait()

    def writeback(c, nb):
        start = pl.multiple_of(base + FIRST + c * CHUNK, 8)
        pltpu.async_copy(
            rows_v.at[nb], out_hbm.at[pl.ds(start, CHUNK)],
            wsem.at[nb])

    def wait_writeback(c, nb):
        start = pl.multiple_of(base + FIRST + c * CHUNK, 8)
        pltpu.make_async_copy(
            rows_v.at[nb], out_hbm.at[pl.ds(start, CHUNK)],
            wsem.at[nb]).wait()

    hw = n_per_w // 2
    pltpu.sync_copy(x_hbm.at[b, pl.ds(off, hw)], idx_v.at[pl.ds(0, hw)])
    pltpu.async_copy(
        table_hbm.at[idx_v.at[pl.ds(0, FIRST)]], first_v, gsem0)
    gather(0, 0)
    gather(1, 1)
    pltpu.sync_copy(x_hbm.at[b, pl.ds(off + hw, hw)],
                    idx_v.at[pl.ds(hw, hw)])

    pltpu.make_async_copy(
        table_hbm.at[idx_v.at[pl.ds(0, FIRST)]], first_v, gsem0).wait()
    _scale_rows(first_v, None, FIRST)
    pltpu.async_copy(first_v, out_hbm.at[pl.ds(base, FIRST)], wsem0)

    def chunk_body(c, _):
        nb = lax.rem(c, NBUF)
        pb = lax.rem(c + NBUF - 1, NBUF)
        prefetch = c + NBUF - 1 < n_ring

        @pl.when(jnp.logical_and(c >= 1, prefetch))
        def _():
            wait_writeback(c - 1, pb)

        @pl.when(prefetch)
        def _():
            gather(c + NBUF - 1, pb)

        wait_gather(nb)
        _scale_rows(rows_v, nb, CHUNK)
        writeback(c, nb)
        return 0

    lax.fori_loop(0, n_ring, chunk_body, 0)

    pltpu.make_async_copy(
        first_v, out_hbm.at[pl.ds(base, FIRST)], wsem0).wait()
    for c in range(n_ring - NBUF, n_ring):
        wait_writeback(c, c % NBUF)


@jax.jit
def kernel(x, table):
    B, L = x.shape
    n = B * L
    n_per_w = n // NW

    mesh = plsc.VectorSubcoreMesh(
        core_axis_name="c", subcore_axis_name="s", num_cores=NC, num_subcores=NS
    )
    out = pl.kernel(
        functools.partial(_embed_kernel, n_per_w, L),
        out_type=jax.ShapeDtypeStruct((n, D_MODEL), jnp.float32),
        mesh=mesh,
        scratch_types=[
            pltpu.VMEM((n_per_w,), jnp.int32),
            pltpu.VMEM((FIRST, D_MODEL), jnp.float32),
            pltpu.VMEM((NBUF, CHUNK, D_MODEL), jnp.float32),
            pltpu.SemaphoreType.DMA,
            pltpu.SemaphoreType.DMA,
            pltpu.SemaphoreType.DMA((NBUF,)),
            pltpu.SemaphoreType.DMA((NBUF,)),
        ],
    )(table, x.astype(jnp.int32))
    return out.reshape(B, L, D_MODEL)


# speedup vs baseline: 1.3045x; 1.0288x over previous
import functools
import math

import jax
import jax.numpy as jnp
from jax import lax
from jax.experimental import pallas as pl
from jax.experimental.pallas import tpu as pltpu
from jax.experimental.pallas import tpu_sc as plsc

D_MODEL = 1024
SCALE = math.sqrt(D_MODEL)
NC, NS, LANES = 2, 16, 16
NW = NC * NS
FIRST = 16
CHUNK = 24
NBUF = 4


def _scale_rows(ref, nb, n_rows):
    @plsc.parallel_loop(0, n_rows)
    def row_body(r):
        @plsc.parallel_loop(0, D_MODEL, LANES, unroll=8)
        def col_body(v):
            sl = pl.ds(v, LANES)
            if nb is None:
                ref[r, sl] = ref[r, sl] * SCALE
            else:
                ref[nb, r, sl] = ref[nb, r, sl] * SCALE


def _embed_kernel(n_per_w, seq_len, table_hbm, x_hbm, out_hbm, idx_v,
                  first_v, rows_v, gsem0, wsem0, gsem, wsem):
    wid = lax.axis_index("s") * NC + lax.axis_index("c")
    base = wid * n_per_w
    w_per_row = seq_len // n_per_w
    b = wid // w_per_row
    off = (wid % w_per_row) * n_per_w

    n_ring = (n_per_w - FIRST) // CHUNK

    def gather(c, nb):
        start = pl.multiple_of(FIRST + c * CHUNK, 8)
        pltpu.async_copy(
            table_hbm.at[idx_v.at[pl.ds(start, CHUNK)]],
            rows_v.at[nb], gsem.at[nb])

    def wait_gather(nb):
        pltpu.make_async_copy(
            table_hbm.at[idx_v.at[pl.ds(0, CHUNK)]],
            rows_v.at[nb], gsem.at[nb]).wait()

    def writeback(c, nb):
        start = pl.multiple_of(base + FIRST + c * CHUNK, 8)
        pltpu.async_copy(
            rows_v.at[nb], out_hbm.at[pl.ds(start, CHUNK)],
            wsem.at[nb])

    def wait_writeback(c, nb):
        start = pl.multiple_of(base + FIRST + c * CHUNK, 8)
        pltpu.make_async_copy(
            rows_v.at[nb], out_hbm.at[pl.ds(start, CHUNK)],
            wsem.at[nb]).wait()

    hw = n_per_w // 2
    pltpu.sync_copy(x_hbm.at[b, pl.ds(off, hw)], idx_v.at[pl.ds(0, hw)])
    pltpu.async_copy(
        table_hbm.at[idx_v.at[pl.ds(0, FIRST)]], first_v, gsem0)
    for c in range(NBUF - 1):
        gather(c, c)
    pltpu.sync_copy(x_hbm.at[b, pl.ds(off + hw, hw)],
                    idx_v.at[pl.ds(hw, hw)])

    pltpu.make_async_copy(
        table_hbm.at[idx_v.at[pl.ds(0, FIRST)]], first_v, gsem0).wait()
    _scale_rows(first_v, None, FIRST)
    pltpu.async_copy(first_v, out_hbm.at[pl.ds(base, FIRST)], wsem0)

    def chunk_body(c, _):
        nb = lax.rem(c, NBUF)
        pb = lax.rem(c + NBUF - 1, NBUF)
        prefetch = c + NBUF - 1 < n_ring

        @pl.when(jnp.logical_and(c >= 1, prefetch))
        def _():
            wait_writeback(c - 1, pb)

        @pl.when(prefetch)
        def _():
            gather(c + NBUF - 1, pb)

        wait_gather(nb)
        _scale_rows(rows_v, nb, CHUNK)
        writeback(c, nb)
        return 0

    lax.fori_loop(0, n_ring, chunk_body, 0)

    pltpu.make_async_copy(
        first_v, out_hbm.at[pl.ds(base, FIRST)], wsem0).wait()
    for c in range(n_ring - NBUF, n_ring):
        wait_writeback(c, c % NBUF)


@jax.jit
def kernel(x, table):
    B, L = x.shape
    n = B * L
    n_per_w = n // NW

    mesh = plsc.VectorSubcoreMesh(
        core_axis_name="c", subcore_axis_name="s", num_cores=NC, num_subcores=NS
    )
    out = pl.kernel(
        functools.partial(_embed_kernel, n_per_w, L),
        out_type=jax.ShapeDtypeStruct((n, D_MODEL), jnp.float32),
        mesh=mesh,
        scratch_types=[
            pltpu.VMEM((n_per_w,), jnp.int32),
            pltpu.VMEM((FIRST, D_MODEL), jnp.float32),
            pltpu.VMEM((NBUF, CHUNK, D_MODEL), jnp.float32),
            pltpu.SemaphoreType.DMA,
            pltpu.SemaphoreType.DMA,
            pltpu.SemaphoreType.DMA((NBUF,)),
            pltpu.SemaphoreType.DMA((NBUF,)),
        ],
    )(table, x.astype(jnp.int32))
    return out.reshape(B, L, D_MODEL)
